# baseline (speedup 1.0000x reference)
import functools
import jax
import jax.numpy as jnp
from jax import lax
from jax.experimental import pallas as pl
from jax.experimental.pallas import tpu as pltpu
from jax.experimental.pallas import tpu_sc as plsc

N = 10000
E = 160000
D = 256
DH = 128
NP = 10240
E_PAD = 163840
CHUNK = 128
SUP = 4
TILE_E = E_PAD // 16
N_CHUNKS = TILE_E // CHUNK
N_SUPERS = N_CHUNKS // SUP
ROWS_PER_TILE = NP // 16
INIT_STEP = 128

_mesh = plsc.VectorSubcoreMesh(core_axis_name="c", subcore_axis_name="s")


@functools.partial(
    pl.kernel,
    out_type=(
        jax.ShapeDtypeStruct((NP, DH), jnp.float32),
        jax.ShapeDtypeStruct((NP, DH), jnp.float32),
    ),
    mesh=_mesh,
    scratch_types=[
        pltpu.VMEM((TILE_E,), jnp.int32),
        [pltpu.VMEM((SUP, CHUNK), jnp.int32) for _ in range(2)],
        [pltpu.VMEM((SUP, CHUNK), jnp.float32) for _ in range(2)],
        [pltpu.VMEM((CHUNK, DH), jnp.float32) for _ in range(2)],
        pltpu.VMEM_SHARED((NP, DH), jnp.float32),
        pltpu.SemaphoreType.DMA,
        pltpu.SemaphoreType.DMA,
        pltpu.SemaphoreType.DMA,
    ],
)
def _sc_aggr(x0, x1, srcp, dstp, ewp, out0, out1,
             src_v, dst_s, ew_s, rows_r,
             accum, sem, dsem, ssem):
    c = lax.axis_index("c")
    s = lax.axis_index("s")

    def run(x_ref, out_ref):
        row0 = s * ROWS_PER_TILE
        e0 = s * TILE_E
        sup0 = s * N_SUPERS

        pltpu.sync_copy(srcp.at[pl.ds(e0, TILE_E)], src_v)

        def init_chunk(i, _):
            r = row0 + i * INIT_STEP
            pltpu.sync_copy(x_ref.at[pl.ds(r, INIT_STEP)], rows_r[0])
            pltpu.sync_copy(rows_r[0], accum.at[pl.ds(r, INIT_STEP)])
            return 0
        lax.fori_loop(0, ROWS_PER_TILE // INIT_STEP, init_chunk, 0)
        plsc.subcore_barrier()

        def start_super(v, sb):
            pltpu.async_copy(dstp.at[pl.ds((sup0 + v) * SUP, SUP)],
                             dst_s[sb], dsem)
            pltpu.async_copy(ewp.at[pl.ds((sup0 + v) * SUP, SUP)],
                             ew_s[sb], dsem)

        def wait_super(v, sb):
            pltpu.make_async_copy(dstp.at[pl.ds((sup0 + v) * SUP, SUP)],
                                  dst_s[sb], dsem).wait()
            pltpu.make_async_copy(ewp.at[pl.ds((sup0 + v) * SUP, SUP)],
                                  ew_s[sb], dsem).wait()

        def gather_desc(i, rp):
            return pltpu.make_async_copy(
                x_ref.at[src_v.at[pl.ds(i * CHUNK, CHUNK)]], rows_r[rp], sem)

        def start_gather(i, rp):
            pltpu.async_copy(x_ref.at[src_v.at[pl.ds(i * CHUNK, CHUNK)]],
                             rows_r[rp], sem)

        def process(i, l, rp, sp, wait_sc, g_next, sup_next):
            buf = rows_r[rp]
            gather_desc(i, rp).wait()
            if wait_sc:
                pl_ = (l - 1) % SUP
                sp_ = sp if l > 0 else 1 - sp
                pltpu.make_async_copy(rows_r[1 - rp],
                                      accum.at[dst_s[sp_].at[pl_]],
                                      ssem).wait()
            if sup_next:
                start_super(i // SUP + 1, 1 - sp)
            if g_next:
                start_gather(i + 1, 1 - rp)
            if l == 0:
                wait_super(i // SUP, sp)

            def scale(g, _):
                wv = ew_s[sp][l, pl.ds(g * 16, 16)]
                for j in range(16):
                    k = g * 16 + j
                    w = jnp.full((16,), wv[j], jnp.float32)
                    for q in range(DH // 16):
                        sl = pl.ds(q * 16, 16)
                        buf[k, sl] = buf[k, sl] * w
                return 0
            lax.fori_loop(0, CHUNK // 16, scale, 0)

            pltpu.async_copy(buf, accum.at[dst_s[sp].at[l]], ssem, add=True)

        start_super(0, 0)
        start_gather(0, 0)
        for i in range(8):
            process(i, i % SUP, i % 2, (i // SUP) % 2,
                    i > 0, True, i % SUP == 0 and i // SUP + 1 < N_SUPERS)

        def body(p, _):
            i0 = 8 + p * 8
            for r in range(8):
                process(i0 + r, r % SUP, r % 2, (r // SUP) % 2,
                        True, True, r % SUP == 0)
            return 0
        lax.fori_loop(0, (N_CHUNKS - 16) // 8, body, 0)

        for r in range(8):
            i = N_CHUNKS - 8 + r
            process(i, r % SUP, r % 2, (r // SUP) % 2,
                    True, i + 1 < N_CHUNKS,
                    r % SUP == 0 and i // SUP + 1 < N_SUPERS)
        pltpu.make_async_copy(rows_r[1], accum.at[dst_s[1].at[SUP - 1]],
                              ssem).wait()
        plsc.subcore_barrier()

        def out_chunk(i, _):
            r = row0 + i * INIT_STEP
            pltpu.sync_copy(accum.at[pl.ds(r, INIT_STEP)], rows_r[0])
            pltpu.sync_copy(rows_r[0], out_ref.at[pl.ds(r, INIT_STEP)])
            return 0
        lax.fori_loop(0, ROWS_PER_TILE // INIT_STEP, out_chunk, 0)

    pl.when(c == 0)(lambda: run(x0, out0))
    pl.when(c == 1)(lambda: run(x1, out1))


def _tc_mlp_body(split_out, relu_out,
                 h0_ref, h1_ref, w1_ref, b1_ref, w2_ref, b2_ref, *outs):
    h = jnp.concatenate([h0_ref[...], h1_ref[...]], axis=1)
    t = jnp.dot(h, w1_ref[...], preferred_element_type=jnp.float32)
    t = jnp.maximum(t + b1_ref[...], 0.0)
    r = jnp.dot(t, w2_ref[...], preferred_element_type=jnp.float32)
    r = r + b2_ref[...]
    if relu_out:
        r = jnp.maximum(r, 0.0)
    if split_out:
        outs[0][...] = r[:, :DH]
        outs[1][...] = r[:, DH:]
    else:
        outs[0][...] = r


def _tc_mlp(h0, h1, W1, b1, W2, b2, *, split_out, relu_out):
    BN = 1024
    grid = (NP // BN,)
    in_specs = [
        pl.BlockSpec((BN, DH), lambda i: (i, 0)),
        pl.BlockSpec((BN, DH), lambda i: (i, 0)),
        pl.BlockSpec((D, D), lambda i: (0, 0)),
        pl.BlockSpec((1, D), lambda i: (0, 0)),
        pl.BlockSpec((D, D), lambda i: (0, 0)),
        pl.BlockSpec((1, D), lambda i: (0, 0)),
    ]
    if split_out:
        out_shape = (
            jax.ShapeDtypeStruct((NP, DH), jnp.float32),
            jax.ShapeDtypeStruct((NP, DH), jnp.float32),
        )
        out_specs = (
            pl.BlockSpec((BN, DH), lambda i: (i, 0)),
            pl.BlockSpec((BN, DH), lambda i: (i, 0)),
        )
    else:
        out_shape = jax.ShapeDtypeStruct((NP, D), jnp.float32)
        out_specs = pl.BlockSpec((BN, D), lambda i: (i, 0))
    return pl.pallas_call(
        functools.partial(_tc_mlp_body, split_out, relu_out),
        grid=grid,
        in_specs=in_specs,
        out_specs=out_specs,
        out_shape=out_shape,
    )(h0, h1, W1, b1.reshape(1, D), W2, b2.reshape(1, D))


@jax.jit
def kernel(x, edge_index, edge_weight, node_type,
           W1_0, b1_0, W2_0, b2_0, W1_1, b1_1, W2_1, b2_1):
    del node_type
    src = edge_index[0]
    dst = edge_index[1]
    epad = E_PAD - E
    srcp = jnp.concatenate([src, jnp.zeros((epad,), jnp.int32)])
    dstp = jnp.concatenate([dst, jnp.zeros((epad,), jnp.int32)])
    ewp = jnp.concatenate([edge_weight, jnp.zeros((epad,), jnp.float32)])
    dstp = dstp.reshape(E_PAD // CHUNK, CHUNK)
    ewp = ewp.reshape(E_PAD // CHUNK, CHUNK)

    xp = jnp.concatenate([x, jnp.zeros((NP - N, D), jnp.float32)], axis=0)
    x0 = xp[:, :DH]
    x1 = xp[:, DH:]

    h0a, h0b = _sc_aggr(x0, x1, srcp, dstp, ewp)
    g0a, g0b = _tc_mlp(h0a, h0b, W1_0, b1_0, W2_0, b2_0,
                       split_out=True, relu_out=True)
    h1a, h1b = _sc_aggr(g0a, g0b, srcp, dstp, ewp)
    out = _tc_mlp(h1a, h1b, W1_1, b1_1, W2_1, b2_1,
                  split_out=False, relu_out=False)
    return out[:N]


# speedup vs baseline: 1.0882x; 1.0882x over previous
import functools
import jax
import jax.numpy as jnp
from jax import lax
from jax.experimental import pallas as pl
from jax.experimental.pallas import tpu as pltpu
from jax.experimental.pallas import tpu_sc as plsc

N = 10000
E = 160000
D = 256
DH = 128
NP = 10240
E_PAD = 163840
CHUNK = 128
TILE_E = E_PAD // 16
N_CHUNKS = TILE_E // CHUNK
ROWS_PER_TILE = NP // 16

_mesh = plsc.VectorSubcoreMesh(core_axis_name="c", subcore_axis_name="s")


@functools.partial(
    pl.kernel,
    out_type=(
        jax.ShapeDtypeStruct((NP, DH), jnp.float32),
        jax.ShapeDtypeStruct((NP, DH), jnp.float32),
    ),
    mesh=_mesh,
    scratch_types=[
        pltpu.VMEM((TILE_E,), jnp.int32),
        pltpu.VMEM((CHUNK,), jnp.int32),
        pltpu.VMEM((CHUNK,), jnp.int32),
        pltpu.VMEM((CHUNK,), jnp.float32),
        pltpu.VMEM((CHUNK,), jnp.float32),
        pltpu.VMEM((CHUNK, DH), jnp.float32),
        pltpu.VMEM((CHUNK, DH), jnp.float32),
        pltpu.VMEM_SHARED((NP, DH), jnp.float32),
        pltpu.SemaphoreType.DMA,
        pltpu.SemaphoreType.DMA,
        pltpu.SemaphoreType.DMA,
    ],
)
def _sc_aggr(x0, x1, srcp, dstp, ewp, out0, out1,
             src_v, dst_a, dst_b, ew_a, ew_b, rows_a, rows_b,
             accum, sem, dsem, ssem):
    c = lax.axis_index("c")
    s = lax.axis_index("s")

    def run(x_ref, out_ref):
        row0 = s * ROWS_PER_TILE
        e0 = s * TILE_E

        pltpu.sync_copy(srcp.at[pl.ds(e0, TILE_E)], src_v)

        pltpu.sync_copy(x_ref.at[pl.ds(row0, ROWS_PER_TILE)],
                        accum.at[pl.ds(row0, ROWS_PER_TILE)])
        plsc.subcore_barrier()

        def start_chunk(i, buf, dbuf, wbuf):
            eb = e0 + i * CHUNK
            pltpu.async_copy(dstp.at[pl.ds(eb, CHUNK)], dbuf, dsem)
            pltpu.async_copy(ewp.at[pl.ds(eb, CHUNK)], wbuf, dsem)
            pltpu.async_copy(x_ref.at[src_v.at[pl.ds(i * CHUNK, CHUNK)]],
                             buf, sem)

        start_chunk(0, rows_a, dst_a, ew_a)

        def process(i, buf, dbuf, wbuf, nbuf, ndbuf, nwbuf,
                    wait_prev_scatter, do_start):
            eb = e0 + i * CHUNK
            pltpu.make_async_copy(x_ref.at[src_v.at[pl.ds(i * CHUNK, CHUNK)]],
                                  buf, sem).wait()
            if wait_prev_scatter:
                pltpu.make_async_copy(nbuf, accum.at[ndbuf], ssem).wait()
            if do_start:
                start_chunk(i + 1, nbuf, ndbuf, nwbuf)
            pltpu.make_async_copy(dstp.at[pl.ds(eb, CHUNK)], dbuf, dsem).wait()
            pltpu.make_async_copy(ewp.at[pl.ds(eb, CHUNK)], wbuf, dsem).wait()

            def scale(g, _):
                wv = wbuf[pl.ds(g * 16, 16)]
                for j in range(16):
                    k = g * 16 + j
                    w = jnp.full((16,), wv[j], jnp.float32)
                    for q in range(DH // 16):
                        sl = pl.ds(q * 16, 16)
                        buf[k, sl] = buf[k, sl] * w
                return 0
            lax.fori_loop(0, CHUNK // 16, scale, 0)

            pltpu.async_copy(buf, accum.at[dbuf], ssem, add=True)

        process(0, rows_a, dst_a, ew_a, rows_b, dst_b, ew_b, False, True)

        def pair(p, _):
            process(2 * p + 1, rows_b, dst_b, ew_b, rows_a, dst_a, ew_a,
                    True, True)
            process(2 * p + 2, rows_a, dst_a, ew_a, rows_b, dst_b, ew_b,
                    True, True)
            return 0
        lax.fori_loop(0, N_CHUNKS // 2 - 1, pair, 0)
        process(N_CHUNKS - 1, rows_b, dst_b, ew_b, rows_a, dst_a, ew_a,
                True, False)
        pltpu.make_async_copy(rows_b, accum.at[dst_b], ssem).wait()
        plsc.subcore_barrier()

        pltpu.sync_copy(accum.at[pl.ds(row0, ROWS_PER_TILE)],
                        out_ref.at[pl.ds(row0, ROWS_PER_TILE)])

    pl.when(c == 0)(lambda: run(x0, out0))
    pl.when(c == 1)(lambda: run(x1, out1))


def _tc_mlp_body(split_out, relu_out,
                 h0_ref, h1_ref, w1_ref, b1_ref, w2_ref, b2_ref, *outs):
    h = jnp.concatenate([h0_ref[...], h1_ref[...]], axis=1)
    t = jnp.dot(h, w1_ref[...], preferred_element_type=jnp.float32)
    t = jnp.maximum(t + b1_ref[...], 0.0)
    r = jnp.dot(t, w2_ref[...], preferred_element_type=jnp.float32)
    r = r + b2_ref[...]
    if relu_out:
        r = jnp.maximum(r, 0.0)
    if split_out:
        outs[0][...] = r[:, :DH]
        outs[1][...] = r[:, DH:]
    else:
        outs[0][...] = r


def _tc_mlp(h0, h1, W1, b1, W2, b2, *, split_out, relu_out):
    BN = 1024
    grid = (NP // BN,)
    in_specs = [
        pl.BlockSpec((BN, DH), lambda i: (i, 0)),
        pl.BlockSpec((BN, DH), lambda i: (i, 0)),
        pl.BlockSpec((D, D), lambda i: (0, 0)),
        pl.BlockSpec((1, D), lambda i: (0, 0)),
        pl.BlockSpec((D, D), lambda i: (0, 0)),
        pl.BlockSpec((1, D), lambda i: (0, 0)),
    ]
    if split_out:
        out_shape = (
            jax.ShapeDtypeStruct((NP, DH), jnp.float32),
            jax.ShapeDtypeStruct((NP, DH), jnp.float32),
        )
        out_specs = (
            pl.BlockSpec((BN, DH), lambda i: (i, 0)),
            pl.BlockSpec((BN, DH), lambda i: (i, 0)),
        )
    else:
        out_shape = jax.ShapeDtypeStruct((NP, D), jnp.float32)
        out_specs = pl.BlockSpec((BN, D), lambda i: (i, 0))
    return pl.pallas_call(
        functools.partial(_tc_mlp_body, split_out, relu_out),
        grid=grid,
        in_specs=in_specs,
        out_specs=out_specs,
        out_shape=out_shape,
    )(h0, h1, W1, b1.reshape(1, D), W2, b2.reshape(1, D))


@jax.jit
def kernel(x, edge_index, edge_weight, node_type,
           W1_0, b1_0, W2_0, b2_0, W1_1, b1_1, W2_1, b2_1):
    del node_type
    src = edge_index[0]
    dst = edge_index[1]
    epad = E_PAD - E
    srcp = jnp.concatenate([src, jnp.zeros((epad,), jnp.int32)])
    dstp = jnp.concatenate([dst, jnp.zeros((epad,), jnp.int32)])
    ewp = jnp.concatenate([edge_weight, jnp.zeros((epad,), jnp.float32)])

    xp = jnp.concatenate([x, jnp.zeros((NP - N, D), jnp.float32)], axis=0)
    x0 = xp[:, :DH]
    x1 = xp[:, DH:]

    h0a, h0b = _sc_aggr(x0, x1, srcp, dstp, ewp)
    g0a, g0b = _tc_mlp(h0a, h0b, W1_0, b1_0, W2_0, b2_0,
                       split_out=True, relu_out=True)
    h1a, h1b = _sc_aggr(g0a, g0b, srcp, dstp, ewp)
    out = _tc_mlp(h1a, h1b, W1_1, b1_1, W2_1, b2_1,
                  split_out=False, relu_out=False)
    return out[:N]
